# Initial kernel scaffold; baseline (speedup 1.0000x reference)
import jax
import jax.numpy as jnp
from jax.experimental import pallas as pl


def kernel(x_experiment, x_material, edge_index_e2m, edge_index_m2e, Win_exp, bin_exp, Win_mat, bin_mat, conv1_e2m_Wsrc, conv1_e2m_Wdst, conv1_e2m_att_src, conv1_e2m_att_dst, conv1_e2m_bias, conv1_m2e_Wsrc, conv1_m2e_Wdst, conv1_m2e_att_src, conv1_m2e_att_dst, conv1_m2e_bias, conv2_e2m_Wsrc, conv2_e2m_Wdst, conv2_e2m_att_src, conv2_e2m_att_dst, conv2_e2m_bias, conv2_m2e_Wsrc, conv2_m2e_Wdst, conv2_m2e_att_src, conv2_m2e_att_dst, conv2_m2e_bias, Wr1, br1, Wr2, br2, Wr3, br3):
    raise NotImplementedError("write your pallas kernel here")


# speedup vs baseline: 106.4368x; 106.4368x over previous
import functools

import jax
import jax.numpy as jnp
from jax import lax
from jax.experimental import pallas as pl
from jax.experimental.pallas import tpu as pltpu
from jax.experimental.pallas import tpu_sc as plsc

N_NODE = 50000
EDG = 800000
D_IN = 128
HID = 64
NH = 4
CC = 16

NC = 2
NS = 16
LANES = 16

SUB = 128
NCHUNK = 408
EPT = SUB * NCHUNK
EPAD = EPT * NS
NACC = 51200
NJUNK = NACC - N_NODE
RPT = NACC // NS
NPT = N_NODE // NS
WCH = 25

EPS = 1e-16
TC_BLK = 400


def _proj_body(xe_ref, xm_ref, we_ref, be_ref, wm_ref, bm_ref, oe_ref, om_ref):
  oe_ref[...] = jnp.dot(xe_ref[...], we_ref[...],
                        preferred_element_type=jnp.float32) + be_ref[...]
  om_ref[...] = jnp.dot(xm_ref[...], wm_ref[...],
                        preferred_element_type=jnp.float32) + bm_ref[...]


def _input_proj(x_exp, x_mat, We, be, Wm, bm):
  n = x_exp.shape[0]
  grid = (n // TC_BLK,)
  return pl.pallas_call(
      _proj_body,
      grid=grid,
      in_specs=[
          pl.BlockSpec((TC_BLK, D_IN), lambda i: (i, 0)),
          pl.BlockSpec((TC_BLK, D_IN), lambda i: (i, 0)),
          pl.BlockSpec((D_IN, HID), lambda i: (0, 0)),
          pl.BlockSpec((1, HID), lambda i: (0, 0)),
          pl.BlockSpec((D_IN, HID), lambda i: (0, 0)),
          pl.BlockSpec((1, HID), lambda i: (0, 0)),
      ],
      out_specs=[
          pl.BlockSpec((TC_BLK, HID), lambda i: (i, 0)),
          pl.BlockSpec((TC_BLK, HID), lambda i: (i, 0)),
      ],
      out_shape=[
          jax.ShapeDtypeStruct((n, HID), jnp.float32),
          jax.ShapeDtypeStruct((n, HID), jnp.float32),
      ],
  )(x_exp, x_mat, We, be.reshape(1, HID), Wm, bm.reshape(1, HID))


def _prep_body(xs_ref, xd_ref, ws_ref, wa_ref, wd_ref, hs_ref, as_ref, ad_ref):
  hs = jnp.dot(xs_ref[...], ws_ref[...], preferred_element_type=jnp.float32)
  hs_ref[0] = hs[:, :HID // 2]
  hs_ref[1] = hs[:, HID // 2:]
  as_ref[...] = jnp.dot(hs, wa_ref[...], preferred_element_type=jnp.float32)
  ad_ref[...] = jnp.dot(xd_ref[...], wd_ref[...],
                        preferred_element_type=jnp.float32)


def _gat_prep(x_src, x_dst, Wsrc, Wa_src, Wd_att):
  n = x_src.shape[0]
  grid = (n // TC_BLK,)
  return pl.pallas_call(
      _prep_body,
      grid=grid,
      in_specs=[
          pl.BlockSpec((TC_BLK, HID), lambda i: (i, 0)),
          pl.BlockSpec((TC_BLK, HID), lambda i: (i, 0)),
          pl.BlockSpec((HID, HID), lambda i: (0, 0)),
          pl.BlockSpec((HID, NH), lambda i: (0, 0)),
          pl.BlockSpec((HID, NH), lambda i: (0, 0)),
      ],
      out_specs=[
          pl.BlockSpec((2, TC_BLK, HID // 2), lambda i: (0, i, 0)),
          pl.BlockSpec((TC_BLK, NH), lambda i: (i, 0)),
          pl.BlockSpec((TC_BLK, NH), lambda i: (i, 0)),
      ],
      out_shape=[
          jax.ShapeDtypeStruct((2, n, HID // 2), jnp.float32),
          jax.ShapeDtypeStruct((n, NH), jnp.float32),
          jax.ShapeDtypeStruct((n, NH), jnp.float32),
      ],
  )(x_src, x_dst, Wsrc, Wa_src, Wd_att)


def _combine_body(f_ref, b_ref, o_ref):
  v = jnp.concatenate([f_ref[0], f_ref[1]], axis=1) + b_ref[...]
  o_ref[...] = jnp.where(v > 0, v, jnp.exp(jnp.minimum(v, 0.0)) - 1.0)


def _combine(outf, bias):
  grid = (N_NODE // TC_BLK,)
  return pl.pallas_call(
      _combine_body,
      grid=grid,
      in_specs=[
          pl.BlockSpec((2, TC_BLK, HID // 2), lambda i: (0, i, 0)),
          pl.BlockSpec((1, HID), lambda i: (0, 0)),
      ],
      out_specs=pl.BlockSpec((TC_BLK, HID), lambda i: (i, 0)),
      out_shape=jax.ShapeDtypeStruct((N_NODE, HID), jnp.float32),
  )(outf, bias.reshape(1, HID))


def _regressor_body(x_ref, w1_ref, b1_ref, w2_ref, b2_ref, w3_ref, b3_ref,
                    o_ref):
  h = jnp.dot(x_ref[...], w1_ref[...], preferred_element_type=jnp.float32)
  h = jnp.maximum(h + b1_ref[...], 0.0)
  h = jnp.dot(h, w2_ref[...], preferred_element_type=jnp.float32)
  h = jnp.maximum(h + b2_ref[...], 0.0)
  o_ref[...] = jnp.dot(h, w3_ref[...],
                       preferred_element_type=jnp.float32) + b3_ref[...]


def _regressor(x, W1, b1, W2, b2, W3, b3):
  grid = (N_NODE // TC_BLK,)
  return pl.pallas_call(
      _regressor_body,
      grid=grid,
      in_specs=[
          pl.BlockSpec((TC_BLK, HID), lambda i: (i, 0)),
          pl.BlockSpec((HID, HID), lambda i: (0, 0)),
          pl.BlockSpec((1, HID), lambda i: (0, 0)),
          pl.BlockSpec((HID, HID // 2), lambda i: (0, 0)),
          pl.BlockSpec((1, HID // 2), lambda i: (0, 0)),
          pl.BlockSpec((HID // 2, 1), lambda i: (0, 0)),
          pl.BlockSpec((1, 1), lambda i: (0, 0)),
      ],
      out_specs=pl.BlockSpec((TC_BLK, 1), lambda i: (i, 0)),
      out_shape=jax.ShapeDtypeStruct((N_NODE, 1), jnp.float32),
  )(x, W1, b1.reshape(1, HID), W2, b2.reshape(1, HID // 2), W3,
    b3.reshape(1, 1))


def _sc_agg_body(s2d, d2d, asf, adf, hstab, zerF, zerD,
                 outf,
                 sidx, didx, bsh, bs0, bs1, bd0, bd1,
                 asr0, asr1, adr0, adr1, ex0, ex1, hsv,
                 accf, accd0, accd1, gsems, ssems):
  cid = lax.axis_index("c")
  tid = lax.axis_index("s")
  wrow0 = tid * NCHUNK
  cidN = cid * N_NODE

  iota = lax.iota(jnp.int32, LANES)
  zero16 = jnp.bitwise_and(iota, 0)

  pltpu.sync_copy(zerF, accf.at[pl.ds(tid * RPT, RPT)])
  pltpu.sync_copy(zerD, accd0.at[pl.ds(tid * RPT, RPT)])
  pltpu.sync_copy(zerD, accd1.at[pl.ds(tid * RPT, RPT)])
  plsc.subcore_barrier()

  def issue_gathers(S, c):
    pltpu.sync_copy(s2d.at[wrow0 + c], sidx.at[S])
    pltpu.sync_copy(d2d.at[wrow0 + c], didx.at[S])
    for m in range(SUB // LANES):
      sl = pl.ds(m * LANES, LANES)
      sv = sidx[S, sl]
      dv = didx[S, sl]
      bsh[S, sl] = sv + cidN
      b0 = sv + (2 * cidN)
      bs0[S, sl] = b0
      bs1[S, sl] = b0 + N_NODE
      d0 = dv + (2 * cid * NACC)
      bd0[S, sl] = d0
      bd1[S, sl] = d0 + NACC
    pltpu.async_copy(asf.at[bs0.at[S]], asr0.at[S], gsems.at[S])
    pltpu.async_copy(asf.at[bs1.at[S]], asr1.at[S], gsems.at[S])
    pltpu.async_copy(adf.at[bd0.at[S]], adr0.at[S], gsems.at[S])
    pltpu.async_copy(adf.at[bd1.at[S]], adr1.at[S], gsems.at[S])
    pltpu.async_copy(hstab.at[bsh.at[S]], hsv.at[S], gsems.at[S])

  def drain_gathers(S):
    pltpu.make_async_copy(asf.at[bs0.at[S]], asr0.at[S], gsems.at[S]).wait()
    pltpu.make_async_copy(asf.at[bs1.at[S]], asr1.at[S], gsems.at[S]).wait()
    pltpu.make_async_copy(adf.at[bd0.at[S]], adr0.at[S], gsems.at[S]).wait()
    pltpu.make_async_copy(adf.at[bd1.at[S]], adr1.at[S], gsems.at[S]).wait()
    pltpu.make_async_copy(hstab.at[bsh.at[S]], hsv.at[S], gsems.at[S]).wait()

  def issue_scatters(S):
    pltpu.async_copy(hsv.at[S], accf.at[didx.at[S]], ssems.at[S], add=True)
    pltpu.async_copy(ex0.at[S], accd0.at[didx.at[S]], ssems.at[S], add=True)
    pltpu.async_copy(ex1.at[S], accd1.at[didx.at[S]], ssems.at[S], add=True)

  def drain_scatters(S):
    pltpu.make_async_copy(hsv.at[S], accf.at[didx.at[S]], ssems.at[S]).wait()
    pltpu.make_async_copy(ex0.at[S], accd0.at[didx.at[S]], ssems.at[S]).wait()
    pltpu.make_async_copy(ex1.at[S], accd1.at[didx.at[S]], ssems.at[S]).wait()

  def compute(S):
    def grp(m, _):
      r0 = m * LANES
      sl = pl.ds(r0, LANES)
      a0 = asr0[S, sl] + adr0[S, sl]
      a0 = jnp.maximum(a0, a0 * 0.2)
      e0v = jnp.exp(a0)
      ex0[S, sl] = e0v
      a1 = asr1[S, sl] + adr1[S, sl]
      a1 = jnp.maximum(a1, a1 * 0.2)
      e1v = jnp.exp(a1)
      ex1[S, sl] = e1v
      for i in range(LANES):
        r = r0 + i
        s0 = jnp.take(e0v, zero16 + i)
        s1 = jnp.take(e1v, zero16 + i)
        hsv[S, r, 0:CC] = hsv[S, r, 0:CC] * s0
        hsv[S, r, CC:2 * CC] = hsv[S, r, CC:2 * CC] * s1
      return ()

    lax.fori_loop(0, SUB // LANES, grp, (), unroll=False)

  issue_gathers(0, 0)

  def outer(g, _):
    c0 = 3 * g
    for p in range(3):
      c = c0 + p + 1
      Snew = (p + 1) % 3

      @pl.when(c < NCHUNK)
      def _():
        @pl.when(c >= 3)
        def _():
          drain_scatters(Snew)
        issue_gathers(Snew, c)

      Sprev = p % 3
      drain_gathers(Sprev)
      compute(Sprev)
      issue_scatters(Sprev)
    return ()

  lax.fori_loop(0, NCHUNK // 3, outer, (), unroll=False)

  for S in range(3):
    drain_scatters(S)

  plsc.subcore_barrier()

  def wchunk(w, _):
    base = tid * RPT + w * SUB
    pltpu.sync_copy(accf.at[pl.ds(base, SUB)], hsv.at[0])
    pltpu.sync_copy(accd0.at[pl.ds(base, SUB)], asr0.at[0])
    pltpu.sync_copy(accd1.at[pl.ds(base, SUB)], asr1.at[0])

    def grp(m, _):
      r0 = m * LANES
      rec0 = 1.0 / (asr0[0, pl.ds(r0, LANES)] + EPS)
      rec1 = 1.0 / (asr1[0, pl.ds(r0, LANES)] + EPS)
      for i in range(LANES):
        r = r0 + i
        s0 = jnp.take(rec0, zero16 + i)
        s1 = jnp.take(rec1, zero16 + i)
        hsv[0, r, 0:CC] = hsv[0, r, 0:CC] * s0
        hsv[0, r, CC:2 * CC] = hsv[0, r, CC:2 * CC] * s1
      return ()

    lax.fori_loop(0, SUB // LANES, grp, (), unroll=False)
    pltpu.sync_copy(hsv.at[0], outf.at[cid, pl.ds(base, SUB)])
    return ()

  lax.fori_loop(0, WCH, wchunk, (), unroll=False)


def _sc_aggregate(s2d, d2d, asf, adf, hs2, zerF, zerD):
  mesh = plsc.VectorSubcoreMesh(core_axis_name="c", subcore_axis_name="s",
                                num_cores=NC, num_subcores=NS)
  f = pl.kernel(
      _sc_agg_body,
      out_type=jax.ShapeDtypeStruct((2, NACC, HID // 2), jnp.float32),
      mesh=mesh,
      compiler_params=pltpu.CompilerParams(use_tc_tiling_on_sc=False),
      scratch_types=[
          pltpu.VMEM((3, SUB), jnp.int32),
          pltpu.VMEM((3, SUB), jnp.int32),
          pltpu.VMEM((3, SUB), jnp.int32),
          pltpu.VMEM((3, SUB), jnp.int32),
          pltpu.VMEM((3, SUB), jnp.int32),
          pltpu.VMEM((3, SUB), jnp.int32),
          pltpu.VMEM((3, SUB), jnp.int32),
          pltpu.VMEM((3, SUB), jnp.float32),
          pltpu.VMEM((3, SUB), jnp.float32),
          pltpu.VMEM((3, SUB), jnp.float32),
          pltpu.VMEM((3, SUB), jnp.float32),
          pltpu.VMEM((3, SUB), jnp.float32),
          pltpu.VMEM((3, SUB), jnp.float32),
          pltpu.VMEM((3, SUB, HID // 2), jnp.float32),
          pltpu.VMEM_SHARED((NACC, HID // 2), jnp.float32),
          pltpu.VMEM_SHARED((NACC,), jnp.float32),
          pltpu.VMEM_SHARED((NACC,), jnp.float32),
          pltpu.SemaphoreType.DMA((3,)),
          pltpu.SemaphoreType.DMA((3,)),
      ],
  )
  return f(s2d, d2d, asf, adf, hs2, zerF, zerD)


def _att_matrix(att):
  return (att[:, :, None] * jnp.eye(NH, dtype=jnp.float32)[:, None, :]
          ).reshape(HID, NH)


def _prep_edges(ei):
  npad = EPAD - EDG
  s = ei[0].astype(jnp.int32)
  d = ei[1].astype(jnp.int32)
  ar = jnp.arange(npad, dtype=jnp.int32)
  s_pad = jnp.concatenate([s, ar % N_NODE])
  d_pad = jnp.concatenate([d, N_NODE + (ar % NJUNK)])
  return s_pad.reshape(EPAD // SUB, SUB), d_pad.reshape(EPAD // SUB, SUB)


def _gat_layer(x_src, x_dst, edges, Wsrc, Wdst, att_src, att_dst, bias):
  s2d, d2d = edges
  Wa_src = _att_matrix(att_src)
  Wd_att = jnp.dot(Wdst, _att_matrix(att_dst))
  hs2, asrc, adst = _gat_prep(x_src, x_dst, Wsrc, Wa_src, Wd_att)
  asf = jnp.transpose(asrc).reshape(4 * N_NODE)
  adf = jnp.pad(jnp.transpose(adst), ((0, 0), (0, NJUNK))).reshape(4 * NACC)
  zerF = jnp.zeros((RPT, HID // 2), jnp.float32)
  zerD = jnp.zeros((RPT,), jnp.float32)
  outf = _sc_aggregate(s2d, d2d, asf, adf,
                       hs2.reshape(2 * N_NODE, HID // 2), zerF, zerD)
  return _combine(outf[:, :N_NODE], bias)


def kernel(x_experiment, x_material, edge_index_e2m, edge_index_m2e,
           Win_exp, bin_exp, Win_mat, bin_mat,
           conv1_e2m_Wsrc, conv1_e2m_Wdst, conv1_e2m_att_src,
           conv1_e2m_att_dst, conv1_e2m_bias,
           conv1_m2e_Wsrc, conv1_m2e_Wdst, conv1_m2e_att_src,
           conv1_m2e_att_dst, conv1_m2e_bias,
           conv2_e2m_Wsrc, conv2_e2m_Wdst, conv2_e2m_att_src,
           conv2_e2m_att_dst, conv2_e2m_bias,
           conv2_m2e_Wsrc, conv2_m2e_Wdst, conv2_m2e_att_src,
           conv2_m2e_att_dst, conv2_m2e_bias,
           Wr1, br1, Wr2, br2, Wr3, br3):
  e1 = _prep_edges(edge_index_e2m)
  e2 = _prep_edges(edge_index_m2e)

  xe, xm = _input_proj(x_experiment, x_material, Win_exp, bin_exp,
                       Win_mat, bin_mat)

  xm1 = _gat_layer(xe, xm, e1, conv1_e2m_Wsrc, conv1_e2m_Wdst,
                   conv1_e2m_att_src, conv1_e2m_att_dst, conv1_e2m_bias)
  xe1 = _gat_layer(xm, xe, e2, conv1_m2e_Wsrc, conv1_m2e_Wdst,
                   conv1_m2e_att_src, conv1_m2e_att_dst, conv1_m2e_bias)

  xm2 = _gat_layer(xe1, xm1, e1, conv2_e2m_Wsrc, conv2_e2m_Wdst,
                   conv2_e2m_att_src, conv2_e2m_att_dst, conv2_e2m_bias)
  xe2 = _gat_layer(xm1, xe1, e2, conv2_m2e_Wsrc, conv2_m2e_Wdst,
                   conv2_m2e_att_src, conv2_m2e_att_dst, conv2_m2e_bias)

  pred = _regressor(xe2, Wr1, br1, Wr2, br2, Wr3, br3).reshape(-1)
  return (pred, xe2, xm2)


# speedup vs baseline: 114.9479x; 1.0800x over previous
import functools

import jax
import jax.numpy as jnp
from jax import lax
from jax.experimental import pallas as pl
from jax.experimental.pallas import tpu as pltpu
from jax.experimental.pallas import tpu_sc as plsc

N_NODE = 50000
EDG = 800000
D_IN = 128
HID = 64
NH = 4
CC = 16

NC = 2
NS = 16
LANES = 16

SUB = 128
NCHUNK = 408
EPT = SUB * NCHUNK
EPAD = EPT * NS
NACC = 51200
NJUNK = NACC - N_NODE
RPT = NACC // NS
NPT = N_NODE // NS
WCH = 25

EPS = 1e-16
TC_BLK = 400


def _elu(v):
  return jnp.where(v > 0, v, jnp.exp(jnp.minimum(v, 0.0)) - 1.0)


def _full(shape):
  return pl.BlockSpec(shape, lambda i: tuple(0 for _ in shape))


_HS_SPEC = pl.BlockSpec((2, TC_BLK, HID // 2), lambda i: (0, i, 0))
_A_SPEC = pl.BlockSpec((TC_BLK, NH), lambda i: (i, 0))
_X_SPEC = pl.BlockSpec((TC_BLK, HID), lambda i: (i, 0))


def _dirs(hs_a, as_a, ad_a, hs_b, as_b, ad_b, xsA, xdA, xsB, xdB, w):
  for (hs_ref, as_ref, ad_ref, xs, xd, ws, wa, wd) in (
      (hs_a, as_a, ad_a, xsA, xdA, w[0], w[1], w[2]),
      (hs_b, as_b, ad_b, xsB, xdB, w[3], w[4], w[5])):
    h = jnp.dot(xs, ws[...], preferred_element_type=jnp.float32)
    hs_ref[0] = h[:, :HID // 2]
    hs_ref[1] = h[:, HID // 2:]
    as_ref[...] = jnp.dot(h, wa[...], preferred_element_type=jnp.float32)
    ad_ref[...] = jnp.dot(xd, wd[...], preferred_element_type=jnp.float32)


def _front_body(xe_ref, xm_ref, we, be, wm, bm, w1, w2, w3, w4, w5, w6,
                hs_a, as_a, ad_a, hs_b, as_b, ad_b):
  xe = jnp.dot(xe_ref[...], we[...],
               preferred_element_type=jnp.float32) + be[...]
  xm = jnp.dot(xm_ref[...], wm[...],
               preferred_element_type=jnp.float32) + bm[...]
  _dirs(hs_a, as_a, ad_a, hs_b, as_b, ad_b, xe, xm, xm, xe,
        (w1, w2, w3, w4, w5, w6))


def _mid_body(fa_ref, ba, fb_ref, bb, w1, w2, w3, w4, w5, w6,
              hs_a, as_a, ad_a, hs_b, as_b, ad_b):
  xm1 = _elu(jnp.concatenate([fa_ref[0], fa_ref[1]], axis=1) + ba[...])
  xe1 = _elu(jnp.concatenate([fb_ref[0], fb_ref[1]], axis=1) + bb[...])
  _dirs(hs_a, as_a, ad_a, hs_b, as_b, ad_b, xe1, xm1, xm1, xe1,
        (w1, w2, w3, w4, w5, w6))


def _back_body(fa_ref, ba, fb_ref, bb, w1, b1, w2, b2, w3, b3,
               xm2_ref, xe2_ref, pred_ref):
  xm2 = _elu(jnp.concatenate([fa_ref[0], fa_ref[1]], axis=1) + ba[...])
  xe2 = _elu(jnp.concatenate([fb_ref[0], fb_ref[1]], axis=1) + bb[...])
  xm2_ref[...] = xm2
  xe2_ref[...] = xe2
  h = jnp.dot(xe2, w1[...], preferred_element_type=jnp.float32)
  h = jnp.maximum(h + b1[...], 0.0)
  h = jnp.dot(h, w2[...], preferred_element_type=jnp.float32)
  h = jnp.maximum(h + b2[...], 0.0)
  pred_ref[...] = jnp.dot(h, w3[...],
                          preferred_element_type=jnp.float32) + b3[...]


_PREP_OUT_SPECS = [_HS_SPEC, _A_SPEC, _A_SPEC, _HS_SPEC, _A_SPEC, _A_SPEC]
_PREP_OUT_SHAPE = [
    jax.ShapeDtypeStruct((2, N_NODE, HID // 2), jnp.float32),
    jax.ShapeDtypeStruct((N_NODE, NH), jnp.float32),
    jax.ShapeDtypeStruct((N_NODE, NH), jnp.float32),
    jax.ShapeDtypeStruct((2, N_NODE, HID // 2), jnp.float32),
    jax.ShapeDtypeStruct((N_NODE, NH), jnp.float32),
    jax.ShapeDtypeStruct((N_NODE, NH), jnp.float32),
]
_GRID = (N_NODE // TC_BLK,)


def _tc_front(x_exp, x_mat, We, be, Wm, bm, wA, wB):
  return pl.pallas_call(
      _front_body,
      grid=_GRID,
      in_specs=[
          pl.BlockSpec((TC_BLK, D_IN), lambda i: (i, 0)),
          pl.BlockSpec((TC_BLK, D_IN), lambda i: (i, 0)),
          _full((D_IN, HID)), _full((1, HID)),
          _full((D_IN, HID)), _full((1, HID)),
          _full((HID, HID)), _full((HID, NH)), _full((HID, NH)),
          _full((HID, HID)), _full((HID, NH)), _full((HID, NH)),
      ],
      out_specs=_PREP_OUT_SPECS,
      out_shape=_PREP_OUT_SHAPE,
  )(x_exp, x_mat, We, be.reshape(1, HID), Wm, bm.reshape(1, HID), *wA, *wB)


def _tc_mid(fA, bA, fB, bB, wA, wB):
  return pl.pallas_call(
      _mid_body,
      grid=_GRID,
      in_specs=[
          _HS_SPEC, _full((1, HID)), _HS_SPEC, _full((1, HID)),
          _full((HID, HID)), _full((HID, NH)), _full((HID, NH)),
          _full((HID, HID)), _full((HID, NH)), _full((HID, NH)),
      ],
      out_specs=_PREP_OUT_SPECS,
      out_shape=_PREP_OUT_SHAPE,
  )(fA, bA.reshape(1, HID), fB, bB.reshape(1, HID), *wA, *wB)


def _tc_back(fA, bA, fB, bB, W1, b1, W2, b2, W3, b3):
  return pl.pallas_call(
      _back_body,
      grid=_GRID,
      in_specs=[
          _HS_SPEC, _full((1, HID)), _HS_SPEC, _full((1, HID)),
          _full((HID, HID)), _full((1, HID)),
          _full((HID, HID // 2)), _full((1, HID // 2)),
          _full((HID // 2, 1)), _full((1, 1)),
      ],
      out_specs=[_X_SPEC, _X_SPEC,
                 pl.BlockSpec((TC_BLK, 1), lambda i: (i, 0))],
      out_shape=[
          jax.ShapeDtypeStruct((N_NODE, HID), jnp.float32),
          jax.ShapeDtypeStruct((N_NODE, HID), jnp.float32),
          jax.ShapeDtypeStruct((N_NODE, 1), jnp.float32),
      ],
  )(fA, bA.reshape(1, HID), fB, bB.reshape(1, HID),
    W1, b1.reshape(1, HID), W2, b2.reshape(1, HID // 2), W3,
    b3.reshape(1, 1))


def _sc_agg_body(s2d, d2d, asf, adf, hstab, zerF, zerD,
                 outf,
                 sidx, didx, bsh, bs0, bs1, bd0, bd1,
                 asr0, asr1, adr0, adr1, ex0, ex1, hsv,
                 accf, accd0, accd1, gsems, ssems):
  cid = lax.axis_index("c")
  tid = lax.axis_index("s")
  wrow0 = tid * NCHUNK
  cidN = cid * N_NODE

  iota = lax.iota(jnp.int32, LANES)
  zero16 = jnp.bitwise_and(iota, 0)

  pltpu.sync_copy(zerF, accf.at[pl.ds(tid * RPT, RPT)])
  pltpu.sync_copy(zerD, accd0.at[pl.ds(tid * RPT, RPT)])
  pltpu.sync_copy(zerD, accd1.at[pl.ds(tid * RPT, RPT)])
  plsc.subcore_barrier()

  def issue_gathers(S, c):
    pltpu.sync_copy(s2d.at[wrow0 + c], sidx.at[S])
    pltpu.sync_copy(d2d.at[wrow0 + c], didx.at[S])
    for m in range(SUB // LANES):
      sl = pl.ds(m * LANES, LANES)
      sv = sidx[S, sl]
      dv = didx[S, sl]
      bsh[S, sl] = sv + cidN
      b0 = sv + (2 * cidN)
      bs0[S, sl] = b0
      bs1[S, sl] = b0 + N_NODE
      d0 = dv + (2 * cid * NACC)
      bd0[S, sl] = d0
      bd1[S, sl] = d0 + NACC
    pltpu.async_copy(asf.at[bs0.at[S]], asr0.at[S], gsems.at[S])
    pltpu.async_copy(asf.at[bs1.at[S]], asr1.at[S], gsems.at[S])
    pltpu.async_copy(adf.at[bd0.at[S]], adr0.at[S], gsems.at[S])
    pltpu.async_copy(adf.at[bd1.at[S]], adr1.at[S], gsems.at[S])
    pltpu.async_copy(hstab.at[bsh.at[S]], hsv.at[S], gsems.at[S])

  def drain_gathers(S):
    pltpu.make_async_copy(asf.at[bs0.at[S]], asr0.at[S], gsems.at[S]).wait()
    pltpu.make_async_copy(asf.at[bs1.at[S]], asr1.at[S], gsems.at[S]).wait()
    pltpu.make_async_copy(adf.at[bd0.at[S]], adr0.at[S], gsems.at[S]).wait()
    pltpu.make_async_copy(adf.at[bd1.at[S]], adr1.at[S], gsems.at[S]).wait()
    pltpu.make_async_copy(hstab.at[bsh.at[S]], hsv.at[S], gsems.at[S]).wait()

  def issue_scatters(S):
    pltpu.async_copy(hsv.at[S], accf.at[didx.at[S]], ssems.at[S], add=True)
    pltpu.async_copy(ex0.at[S], accd0.at[didx.at[S]], ssems.at[S], add=True)
    pltpu.async_copy(ex1.at[S], accd1.at[didx.at[S]], ssems.at[S], add=True)

  def drain_scatters(S):
    pltpu.make_async_copy(hsv.at[S], accf.at[didx.at[S]], ssems.at[S]).wait()
    pltpu.make_async_copy(ex0.at[S], accd0.at[didx.at[S]], ssems.at[S]).wait()
    pltpu.make_async_copy(ex1.at[S], accd1.at[didx.at[S]], ssems.at[S]).wait()

  def compute(S):
    def grp(m, _):
      r0 = m * LANES
      sl = pl.ds(r0, LANES)
      a0 = asr0[S, sl] + adr0[S, sl]
      a0 = jnp.maximum(a0, a0 * 0.2)
      e0v = jnp.exp(a0)
      ex0[S, sl] = e0v
      a1 = asr1[S, sl] + adr1[S, sl]
      a1 = jnp.maximum(a1, a1 * 0.2)
      e1v = jnp.exp(a1)
      ex1[S, sl] = e1v
      for i in range(LANES):
        r = r0 + i
        s0 = jnp.take(e0v, zero16 + i)
        s1 = jnp.take(e1v, zero16 + i)
        hsv[S, r, 0:CC] = hsv[S, r, 0:CC] * s0
        hsv[S, r, CC:2 * CC] = hsv[S, r, CC:2 * CC] * s1
      return ()

    lax.fori_loop(0, SUB // LANES, grp, (), unroll=False)

  issue_gathers(0, 0)

  def outer(g, _):
    c0 = 3 * g
    for p in range(3):
      c = c0 + p + 1
      Snew = (p + 1) % 3

      @pl.when(c < NCHUNK)
      def _():
        @pl.when(c >= 3)
        def _():
          drain_scatters(Snew)
        issue_gathers(Snew, c)

      Sprev = p % 3
      drain_gathers(Sprev)
      compute(Sprev)
      issue_scatters(Sprev)
    return ()

  lax.fori_loop(0, NCHUNK // 3, outer, (), unroll=False)

  for S in range(3):
    drain_scatters(S)

  plsc.subcore_barrier()

  def wchunk(w, _):
    base = tid * RPT + w * SUB
    pltpu.sync_copy(accf.at[pl.ds(base, SUB)], hsv.at[0])
    pltpu.sync_copy(accd0.at[pl.ds(base, SUB)], asr0.at[0])
    pltpu.sync_copy(accd1.at[pl.ds(base, SUB)], asr1.at[0])

    def grp(m, _):
      r0 = m * LANES
      rec0 = 1.0 / (asr0[0, pl.ds(r0, LANES)] + EPS)
      rec1 = 1.0 / (asr1[0, pl.ds(r0, LANES)] + EPS)
      for i in range(LANES):
        r = r0 + i
        s0 = jnp.take(rec0, zero16 + i)
        s1 = jnp.take(rec1, zero16 + i)
        hsv[0, r, 0:CC] = hsv[0, r, 0:CC] * s0
        hsv[0, r, CC:2 * CC] = hsv[0, r, CC:2 * CC] * s1
      return ()

    lax.fori_loop(0, SUB // LANES, grp, (), unroll=False)
    pltpu.sync_copy(hsv.at[0], outf.at[cid, pl.ds(base, SUB)])
    return ()

  lax.fori_loop(0, WCH, wchunk, (), unroll=False)


def _sc_aggregate(s2d, d2d, asf, adf, hs2, zerF, zerD):
  mesh = plsc.VectorSubcoreMesh(core_axis_name="c", subcore_axis_name="s",
                                num_cores=NC, num_subcores=NS)
  f = pl.kernel(
      _sc_agg_body,
      out_type=jax.ShapeDtypeStruct((2, NACC, HID // 2), jnp.float32),
      mesh=mesh,
      compiler_params=pltpu.CompilerParams(use_tc_tiling_on_sc=False),
      scratch_types=[
          pltpu.VMEM((3, SUB), jnp.int32),
          pltpu.VMEM((3, SUB), jnp.int32),
          pltpu.VMEM((3, SUB), jnp.int32),
          pltpu.VMEM((3, SUB), jnp.int32),
          pltpu.VMEM((3, SUB), jnp.int32),
          pltpu.VMEM((3, SUB), jnp.int32),
          pltpu.VMEM((3, SUB), jnp.int32),
          pltpu.VMEM((3, SUB), jnp.float32),
          pltpu.VMEM((3, SUB), jnp.float32),
          pltpu.VMEM((3, SUB), jnp.float32),
          pltpu.VMEM((3, SUB), jnp.float32),
          pltpu.VMEM((3, SUB), jnp.float32),
          pltpu.VMEM((3, SUB), jnp.float32),
          pltpu.VMEM((3, SUB, HID // 2), jnp.float32),
          pltpu.VMEM_SHARED((NACC, HID // 2), jnp.float32),
          pltpu.VMEM_SHARED((NACC,), jnp.float32),
          pltpu.VMEM_SHARED((NACC,), jnp.float32),
          pltpu.SemaphoreType.DMA((3,)),
          pltpu.SemaphoreType.DMA((3,)),
      ],
  )
  return f(s2d, d2d, asf, adf, hs2, zerF, zerD)


def _att_matrix(att):
  return (att[:, :, None] * jnp.eye(NH, dtype=jnp.float32)[:, None, :]
          ).reshape(HID, NH)


def _prep_edges(ei):
  npad = EPAD - EDG
  s = ei[0].astype(jnp.int32)
  d = ei[1].astype(jnp.int32)
  ar = jnp.arange(npad, dtype=jnp.int32)
  s_pad = jnp.concatenate([s, ar % N_NODE])
  d_pad = jnp.concatenate([d, N_NODE + (ar % NJUNK)])
  return s_pad.reshape(EPAD // SUB, SUB), d_pad.reshape(EPAD // SUB, SUB)


def _agg(edges, hs2, asrc, adst, zerF, zerD):
  s2d, d2d = edges
  asf = jnp.transpose(asrc).reshape(4 * N_NODE)
  adf = jnp.pad(jnp.transpose(adst), ((0, 0), (0, NJUNK))).reshape(4 * NACC)
  outf = _sc_aggregate(s2d, d2d, asf, adf,
                       hs2.reshape(2 * N_NODE, HID // 2), zerF, zerD)
  return outf[:, :N_NODE]


def kernel(x_experiment, x_material, edge_index_e2m, edge_index_m2e,
           Win_exp, bin_exp, Win_mat, bin_mat,
           conv1_e2m_Wsrc, conv1_e2m_Wdst, conv1_e2m_att_src,
           conv1_e2m_att_dst, conv1_e2m_bias,
           conv1_m2e_Wsrc, conv1_m2e_Wdst, conv1_m2e_att_src,
           conv1_m2e_att_dst, conv1_m2e_bias,
           conv2_e2m_Wsrc, conv2_e2m_Wdst, conv2_e2m_att_src,
           conv2_e2m_att_dst, conv2_e2m_bias,
           conv2_m2e_Wsrc, conv2_m2e_Wdst, conv2_m2e_att_src,
           conv2_m2e_att_dst, conv2_m2e_bias,
           Wr1, br1, Wr2, br2, Wr3, br3):
  e1 = _prep_edges(edge_index_e2m)
  e2 = _prep_edges(edge_index_m2e)
  zerF = jnp.zeros((RPT, HID // 2), jnp.float32)
  zerD = jnp.zeros((RPT,), jnp.float32)

  def fold(Wsrc, att_src, Wdst, att_dst):
    return (Wsrc, _att_matrix(att_src), jnp.dot(Wdst, _att_matrix(att_dst)))

  wA1 = fold(conv1_e2m_Wsrc, conv1_e2m_att_src,
             conv1_e2m_Wdst, conv1_e2m_att_dst)
  wB1 = fold(conv1_m2e_Wsrc, conv1_m2e_att_src,
             conv1_m2e_Wdst, conv1_m2e_att_dst)
  wA2 = fold(conv2_e2m_Wsrc, conv2_e2m_att_src,
             conv2_e2m_Wdst, conv2_e2m_att_dst)
  wB2 = fold(conv2_m2e_Wsrc, conv2_m2e_att_src,
             conv2_m2e_Wdst, conv2_m2e_att_dst)

  hsA, asA, adA, hsB, asB, adB = _tc_front(
      x_experiment, x_material, Win_exp, bin_exp, Win_mat, bin_mat, wA1, wB1)
  fA = _agg(e1, hsA, asA, adA, zerF, zerD)
  fB = _agg(e2, hsB, asB, adB, zerF, zerD)

  hsA2, asA2, adA2, hsB2, asB2, adB2 = _tc_mid(
      fA, conv1_e2m_bias, fB, conv1_m2e_bias, wA2, wB2)
  fA2 = _agg(e1, hsA2, asA2, adA2, zerF, zerD)
  fB2 = _agg(e2, hsB2, asB2, adB2, zerF, zerD)

  xm2, xe2, pred = _tc_back(fA2, conv2_e2m_bias, fB2, conv2_m2e_bias,
                            Wr1, br1, Wr2, br2, Wr3, br3)
  return (pred.reshape(-1), xe2, xm2)
